# Initial kernel scaffold; baseline (speedup 1.0000x reference)
import jax
import jax.numpy as jnp
from jax.experimental import pallas as pl


def kernel(input_ids, token_table, pos_table, gamma, beta):
    raise NotImplementedError("write your pallas kernel here")


# speedup vs baseline: 4.0058x; 4.0058x over previous
import jax
import jax.numpy as jnp
from jax.experimental import pallas as pl
from jax.experimental.pallas import tpu as pltpu

B, L, H, VOCAB = 128, 512, 768, 8


def _embed_ln_kernel(ids_ref, tok_ref, pos_ref, gamma_ref, beta_ref, out_ref):
    ids = ids_ref[0, 0, :]
    iota = jax.lax.broadcasted_iota(jnp.int32, (L, VOCAB), 1)
    onehot = (ids[:, None] == iota).astype(jnp.float32)
    tok = jnp.dot(onehot, tok_ref[...], preferred_element_type=jnp.float32)
    emb = tok + pos_ref[...]
    mean = jnp.mean(emb, axis=1, keepdims=True)
    c = emb - mean
    var = jnp.mean(c * c, axis=1, keepdims=True)
    normed = c * jax.lax.rsqrt(var + 1e-5)
    out_ref[0] = normed * gamma_ref[0] + beta_ref[0]


def kernel(input_ids, token_table, pos_table, gamma, beta):
    ids3 = input_ids.reshape(B, 1, L).astype(jnp.int32)
    gamma2 = gamma.reshape(1, H)
    beta2 = beta.reshape(1, H)
    out = pl.pallas_call(
        _embed_ln_kernel,
        grid=(B,),
        in_specs=[
            pl.BlockSpec((1, 1, L), lambda i: (i, 0, 0)),
            pl.BlockSpec((VOCAB, H), lambda i: (0, 0)),
            pl.BlockSpec((L, H), lambda i: (0, 0)),
            pl.BlockSpec((1, H), lambda i: (0, 0)),
            pl.BlockSpec((1, H), lambda i: (0, 0)),
        ],
        out_specs=pl.BlockSpec((1, L, H), lambda i: (i, 0, 0)),
        out_shape=jax.ShapeDtypeStruct((B, L, H), jnp.float32),
        compiler_params=pltpu.CompilerParams(
            dimension_semantics=("arbitrary",),
        ),
    )(ids3, token_table, pos_table, gamma2, beta2)
    return out


# speedup vs baseline: 4.2524x; 1.0616x over previous
import jax
import jax.numpy as jnp
from jax.experimental import pallas as pl
from jax.experimental.pallas import tpu as pltpu

B, L, H, VOCAB = 128, 512, 768, 8
EPS = 1e-5


def _stats_kernel(tok_ref, pos_ref, rstd_ref, rm_ref):
    tok = tok_ref[...]
    pos = pos_ref[...]
    inv_h = 1.0 / H
    ones_row = jnp.ones((1, H), dtype=jnp.float32)
    mp = jnp.mean(pos, axis=1, keepdims=True)
    ep2 = jnp.mean(pos * pos, axis=1, keepdims=True)
    mt = jax.lax.dot_general(ones_row, tok, (((1,), (1,)), ((), ())),
                             preferred_element_type=jnp.float32) * inv_h
    et2 = jax.lax.dot_general(ones_row, tok * tok, (((1,), (1,)), ((), ())),
                              preferred_element_type=jnp.float32) * inv_h
    cross = jax.lax.dot_general(pos, tok, (((1,), (1,)), ((), ())),
                                preferred_element_type=jnp.float32) * inv_h
    mu = mp + mt
    var = ep2 + et2 + 2.0 * cross - mu * mu
    rstd = jax.lax.rsqrt(var + EPS)
    rstd_ref[...] = rstd
    rm_ref[...] = rstd * mu


def _embed_ln_kernel(ids_ref, tok_ref, pos_ref, rstd_ref, rm_ref,
                     gamma_ref, beta_ref, out_ref):
    ids = ids_ref[0, 0, :]
    iota = jax.lax.broadcasted_iota(jnp.int32, (L, VOCAB), 1)
    onehot = (ids[:, None] == iota).astype(jnp.float32)
    tok = jnp.dot(onehot, tok_ref[...], preferred_element_type=jnp.float32)
    a = jnp.sum(onehot * rstd_ref[...], axis=1, keepdims=True)
    s = jnp.sum(onehot * rm_ref[...], axis=1, keepdims=True)
    t = (tok + pos_ref[...]) * a - s
    out_ref[0] = t * gamma_ref[0] + beta_ref[0]


def kernel(input_ids, token_table, pos_table, gamma, beta):
    rstd, rm = pl.pallas_call(
        _stats_kernel,
        out_shape=(
            jax.ShapeDtypeStruct((L, VOCAB), jnp.float32),
            jax.ShapeDtypeStruct((L, VOCAB), jnp.float32),
        ),
    )(token_table, pos_table)

    ids3 = input_ids.reshape(B, 1, L).astype(jnp.int32)
    gamma2 = gamma.reshape(1, H)
    beta2 = beta.reshape(1, H)
    out = pl.pallas_call(
        _embed_ln_kernel,
        grid=(B,),
        in_specs=[
            pl.BlockSpec((1, 1, L), lambda i: (i, 0, 0)),
            pl.BlockSpec((VOCAB, H), lambda i: (0, 0)),
            pl.BlockSpec((L, H), lambda i: (0, 0)),
            pl.BlockSpec((L, VOCAB), lambda i: (0, 0)),
            pl.BlockSpec((L, VOCAB), lambda i: (0, 0)),
            pl.BlockSpec((1, H), lambda i: (0, 0)),
            pl.BlockSpec((1, H), lambda i: (0, 0)),
        ],
        out_specs=pl.BlockSpec((1, L, H), lambda i: (i, 0, 0)),
        out_shape=jax.ShapeDtypeStruct((B, L, H), jnp.float32),
        compiler_params=pltpu.CompilerParams(
            dimension_semantics=("arbitrary",),
        ),
    )(ids3, token_table, pos_table, rstd, rm, gamma2, beta2)
    return out


# speedup vs baseline: 5.9726x; 1.4045x over previous
import jax
import jax.numpy as jnp
from jax.experimental import pallas as pl
from jax.experimental.pallas import tpu as pltpu

B, L, H, VOCAB = 128, 512, 768, 8
EPS = 1e-5


def _stats_kernel(tok_ref, pos_ref, rstd_ref, rm_ref):
    tok = tok_ref[...]
    pos = pos_ref[...]
    inv_h = 1.0 / H
    ones_row = jnp.ones((1, H), dtype=jnp.float32)
    mp = jnp.mean(pos, axis=1, keepdims=True)
    ep2 = jnp.mean(pos * pos, axis=1, keepdims=True)
    mt = jax.lax.dot_general(ones_row, tok, (((1,), (1,)), ((), ())),
                             preferred_element_type=jnp.float32) * inv_h
    et2 = jax.lax.dot_general(ones_row, tok * tok, (((1,), (1,)), ((), ())),
                              preferred_element_type=jnp.float32) * inv_h
    cross = jax.lax.dot_general(pos, tok, (((1,), (1,)), ((), ())),
                                preferred_element_type=jnp.float32) * inv_h
    mu = mp + mt
    var = ep2 + et2 + 2.0 * cross - mu * mu
    rstd = jax.lax.rsqrt(var + EPS)
    rstd_ref[...] = rstd
    rm_ref[...] = rstd * mu


ROWS = 4


def _embed_ln_kernel(ids_ref, tok_ref, pos_ref, rstd_ref, rm_ref,
                     gamma_ref, beta_ref, out_ref):
    for r in range(ROWS):
        ids = ids_ref[r, 0, :]
        iota = jax.lax.broadcasted_iota(jnp.int32, (L, VOCAB), 1)
        onehot = (ids[:, None] == iota).astype(jnp.float32)
        tok = jnp.dot(onehot, tok_ref[...], preferred_element_type=jnp.float32)
        a = jnp.sum(onehot * rstd_ref[...], axis=1, keepdims=True)
        s = jnp.sum(onehot * rm_ref[...], axis=1, keepdims=True)
        t = (tok + pos_ref[...]) * a - s
        out_ref[r] = t * gamma_ref[0] + beta_ref[0]


def kernel(input_ids, token_table, pos_table, gamma, beta):
    rstd, rm = pl.pallas_call(
        _stats_kernel,
        out_shape=(
            jax.ShapeDtypeStruct((L, VOCAB), jnp.float32),
            jax.ShapeDtypeStruct((L, VOCAB), jnp.float32),
        ),
    )(token_table, pos_table)

    ids3 = input_ids.reshape(B, 1, L).astype(jnp.int32)
    gamma2 = gamma.reshape(1, H)
    beta2 = beta.reshape(1, H)
    out = pl.pallas_call(
        _embed_ln_kernel,
        grid=(B // ROWS,),
        in_specs=[
            pl.BlockSpec((ROWS, 1, L), lambda i: (i, 0, 0)),
            pl.BlockSpec((VOCAB, H), lambda i: (0, 0)),
            pl.BlockSpec((L, H), lambda i: (0, 0)),
            pl.BlockSpec((L, VOCAB), lambda i: (0, 0)),
            pl.BlockSpec((L, VOCAB), lambda i: (0, 0)),
            pl.BlockSpec((1, H), lambda i: (0, 0)),
            pl.BlockSpec((1, H), lambda i: (0, 0)),
        ],
        out_specs=pl.BlockSpec((ROWS, L, H), lambda i: (i, 0, 0)),
        out_shape=jax.ShapeDtypeStruct((B, L, H), jnp.float32),
        compiler_params=pltpu.CompilerParams(
            dimension_semantics=("arbitrary",),
        ),
    )(ids3, token_table, pos_table, rstd, rm, gamma2, beta2)
    return out
